# Initial kernel scaffold; baseline (speedup 1.0000x reference)
import jax
import jax.numpy as jnp
from jax.experimental import pallas as pl


def kernel(x, edge_index1, edge_index2, W1_l, W1_r, b1, W2_l, W2_r, b2, size1, size2):
    raise NotImplementedError("write your pallas kernel here")


# speedup vs baseline: 5.5641x; 5.5641x over previous
import functools

import jax
import jax.numpy as jnp
from jax import lax
from jax.experimental import pallas as pl
from jax.experimental.pallas import tpu as pltpu
from jax.experimental.pallas import tpu_sc as plsc

_T1, _T2 = 5000, 2500
_D = 128
_C = 47
_W1 = 144
_W2 = 48
_NW = 32
_B = 128
_ACC1 = 5120
_ACC2 = 2560


def _make_seg_sum(n_chunks: int, width: int, acc_rows: int):
    mesh = plsc.VectorSubcoreMesh(core_axis_name="c", subcore_axis_name="s")
    rows_per_sub = acc_rows // 16

    @functools.partial(
        pl.kernel,
        out_type=jax.ShapeDtypeStruct((2, acc_rows, width), jnp.float32),
        mesh=mesh,
        scratch_types=[
            pltpu.VMEM((n_chunks, _B), jnp.int32),
            pltpu.VMEM((n_chunks, _B), jnp.int32),
            pltpu.VMEM((_B, width), jnp.float32),
            pltpu.VMEM((16, width), jnp.float32),
            pltpu.VMEM_SHARED((acc_rows, width), jnp.float32),
            pltpu.SemaphoreType.DMA,
        ],
        compiler_params=pltpu.CompilerParams(use_tc_tiling_on_sc=False),
    )
    def seg_sum(table_hbm, src_hbm, dst_hbm, out_hbm,
                src_v, dst_v, rows_v, zbuf, acc_sh, sem):
        c = lax.axis_index("c")
        s = lax.axis_index("s")
        wid = s * 2 + c

        for r in range(16):
            for col in range(width // 16):
                zbuf[r, pl.ds(col * 16, 16)] = jnp.zeros((16,), jnp.float32)

        def zero_body(j, carry):
            pltpu.sync_copy(zbuf, acc_sh.at[pl.ds(s * rows_per_sub + j * 16, 16)])
            return carry
        lax.fori_loop(0, rows_per_sub // 16, zero_body, 0)
        plsc.subcore_barrier()

        pltpu.sync_copy(src_hbm.at[wid], src_v)
        pltpu.sync_copy(dst_hbm.at[wid], dst_v)

        def body(j, carry):
            pltpu.async_copy(table_hbm.at[src_v.at[j]], rows_v, sem).wait()
            pltpu.sync_copy(rows_v, acc_sh.at[dst_v.at[j]], add=True)
            return carry
        lax.fori_loop(0, n_chunks, body, 0)

        plsc.subcore_barrier()
        pltpu.sync_copy(acc_sh.at[pl.ds(s * rows_per_sub, rows_per_sub)],
                        out_hbm.at[c, pl.ds(s * rows_per_sub, rows_per_sub)])

    return seg_sum


def _tc_layer1(parts, x_t, w1l, w1r, b1e, w2pad):
    def body(parts_ref, xt_ref, w1l_ref, w1r_ref, b1_ref, w2_ref,
             hg_ref, hh_ref):
        agg = parts_ref[0] + parts_ref[1]
        cnt = agg[:_T1, _D:_D + 1]
        mean = agg[:_T1, :_D] / jnp.maximum(cnt, 1.0)
        h = mean @ w1l_ref[...] + xt_ref[...] @ w1r_ref[...] + b1_ref[...]
        h = jnp.maximum(h, 0.0)
        hg = h @ w2_ref[...]
        lane = lax.broadcasted_iota(jnp.int32, hg.shape, 1)
        hg_ref[...] = jnp.where(lane == _C, 1.0, hg)
        hh_ref[...] = h[:_T2, :]

    return pl.pallas_call(
        body,
        out_shape=(
            jax.ShapeDtypeStruct((_T1, _W2), jnp.float32),
            jax.ShapeDtypeStruct((_T2, _D), jnp.float32),
        ),
    )(parts, x_t, w1l, w1r, b1e, w2pad)


def _tc_layer2(parts2, h_half, w2r, b2e):
    def body(parts_ref, hh_ref, w2r_ref, b2_ref, out_ref):
        agg = parts_ref[0] + parts_ref[1]
        cnt = agg[:_T2, _C + 1 - 1:_C + 1]
        meanw = agg[:_T2, :_C] / jnp.maximum(cnt, 1.0)
        z = meanw + hh_ref[...] @ w2r_ref[...] + b2_ref[...]
        m = jnp.max(z, axis=-1, keepdims=True)
        e = jnp.exp(z - m)
        lse = m + jnp.log(jnp.sum(e, axis=-1, keepdims=True))
        out_ref[...] = z - lse

    return pl.pallas_call(
        body,
        out_shape=jax.ShapeDtypeStruct((_T2, _C), jnp.float32),
    )(parts2, h_half, w2r, b2e)


def _pad_edges(src, dst, n_chunks, dummy_row):
    e = src.shape[0]
    total = _NW * n_chunks * _B
    pad = total - e
    src_p = jnp.concatenate([src, jnp.zeros((pad,), jnp.int32)])
    dst_p = jnp.concatenate([dst, jnp.full((pad,), dummy_row, jnp.int32)])
    return (src_p.reshape(_NW, n_chunks, _B), dst_p.reshape(_NW, n_chunks, _B))


def kernel(x, edge_index1, edge_index2, W1_l, W1_r, b1, W2_l, W2_r, b2,
           size1, size2):
    n = x.shape[0]
    e1 = edge_index1.shape[1]
    e2 = edge_index2.shape[1]
    ch1 = -(-e1 // (_NW * _B))
    ch2 = -(-e2 // (_NW * _B))

    dev1 = (jnp.asarray(size1, jnp.int32) - _T1).astype(jnp.float32)
    dev2 = (jnp.asarray(size2, jnp.int32) - _T2).astype(jnp.float32)
    b1e = (b1 + dev1).reshape(1, _D).astype(jnp.float32)
    b2e = (b2 + dev2).reshape(1, _C).astype(jnp.float32)

    x_aug = jnp.concatenate(
        [x, jnp.ones((n, 1), jnp.float32), jnp.zeros((n, _W1 - _D - 1), jnp.float32)],
        axis=1)
    w2pad = jnp.concatenate([W2_l, jnp.zeros((_D, _W2 - _C), jnp.float32)], axis=1)

    src1, dst1 = _pad_edges(edge_index1[0], edge_index1[1], ch1, _T1)
    src2, dst2 = _pad_edges(edge_index2[0], edge_index2[1], ch2, _T2)

    parts1 = _make_seg_sum(ch1, _W1, _ACC1)(x_aug, src1, dst1)
    hg_aug, h_half = _tc_layer1(parts1, x[:_T1], W1_l, W1_r, b1e, w2pad)
    parts2 = _make_seg_sum(ch2, _W2, _ACC2)(hg_aug, src2, dst2)
    return _tc_layer2(parts2, h_half, W2_r, b2e)
